# Initial kernel scaffold; baseline (speedup 1.0000x reference)
import jax
import jax.numpy as jnp
from jax.experimental import pallas as pl


def kernel(x):
    raise NotImplementedError("write your pallas kernel here")


# speedup vs baseline: 1.0131x; 1.0131x over previous
import functools

import jax
import jax.numpy as jnp
from jax.experimental import pallas as pl
from jax.experimental.pallas import tpu as pltpu

_NUM_MASKS = 4
_MASK_LEN = 128
_BLK = 128
_TT = 1024


def _copy_energy_body(x_ref, out_ref, e_ref):
    v = x_ref[...]
    out_ref[...] = v
    e_ref[0, 0, :] = jnp.sum(v[0] * v[0], axis=-1)


def _argmax_body(e_ref, n_ref, s_ref):
    B, T = e_ref.shape
    R = n_ref.shape[0]
    e16 = jnp.broadcast_to(e_ref[...][:, None, :], (B, R // B, T)).reshape(R, T)
    v = e16 + n_ref[...]
    m = jnp.max(v, axis=-1, keepdims=True)
    ii = jax.lax.broadcasted_iota(jnp.int32, v.shape, 1)
    s_ref[0, :] = jnp.min(jnp.where(v == m, ii, T), axis=-1)


def _mask_body(starts_ref, out_ref, o_ref):
    T = 8192
    b = pl.program_id(0)
    i = pl.program_id(1)
    j = pl.program_id(2)
    s_i = starts_ref[b * _NUM_MASKS + i]
    row0 = jnp.minimum(s_i // _BLK + j, T // _BLK - 1) * _BLK
    rows = row0 + jax.lax.broadcasted_iota(jnp.int32, (_BLK, 1), 0)
    mask = jnp.zeros((_BLK, 1), jnp.bool_)
    for k in range(_NUM_MASKS):
        sk = starts_ref[b * _NUM_MASKS + k]
        mask = mask | ((rows >= sk) & (rows < sk + _MASK_LEN))
    o_ref[...] = jnp.where(mask[None], jnp.float32(0.0), out_ref[...])


def _mask_index(b, i, j, starts):
    T = 8192
    row = jnp.minimum(starts[b * _NUM_MASKS + i] // _BLK + j, T // _BLK - 1)
    return (b, row, 0)


@functools.partial(jax.jit, static_argnames=("interpret",))
def kernel(x, interpret=False):
    B, T, C = x.shape

    out, energy3 = pl.pallas_call(
        _copy_energy_body,
        grid=(B, T // _TT),
        in_specs=[pl.BlockSpec((1, _TT, C), lambda b, t: (b, t, 0))],
        out_specs=[pl.BlockSpec((1, _TT, C), lambda b, t: (b, t, 0)),
                   pl.BlockSpec((1, 1, _TT), lambda b, t: (b, 0, t))],
        out_shape=[jax.ShapeDtypeStruct((B, T, C), x.dtype),
                   jax.ShapeDtypeStruct((B, 1, T), jnp.float32)],
        interpret=interpret,
    )(x)
    energy = energy3.reshape(B, T)

    skey = jax.random.key(42)
    noise = jnp.stack(
        [jax.random.gumbel(jax.random.fold_in(skey, i), (B, T), jnp.float32)
         for i in range(_NUM_MASKS)], axis=1).reshape(B * _NUM_MASKS, T)

    starts = pl.pallas_call(
        _argmax_body,
        out_shape=jax.ShapeDtypeStruct((1, B * _NUM_MASKS), jnp.int32),
        interpret=interpret,
    )(energy, noise)
    starts_flat = starts.reshape(B * _NUM_MASKS)

    out2 = pl.pallas_call(
        _mask_body,
        grid_spec=pltpu.PrefetchScalarGridSpec(
            num_scalar_prefetch=1,
            grid=(B, _NUM_MASKS, 2),
            in_specs=[pl.BlockSpec((1, _BLK, C), _mask_index)],
            out_specs=pl.BlockSpec((1, _BLK, C), _mask_index),
        ),
        out_shape=jax.ShapeDtypeStruct((B, T, C), x.dtype),
        input_output_aliases={1: 0},
        interpret=interpret,
    )(starts_flat, out)
    return out2


# speedup vs baseline: 1.0750x; 1.0611x over previous
import functools

import jax
import jax.numpy as jnp
from jax import lax
from jax.experimental import pallas as pl
from jax.experimental.pallas import tpu as pltpu
from jax.experimental.pallas import tpu_sc as plsc

_NUM_MASKS = 4
_MASK_LEN = 128
_BLK = 128
_TT = 2048


def _copy_energy_body(x_ref, out_ref, e_ref):
    v = x_ref[...]
    out_ref[...] = v
    e_ref[0, 0, :] = jnp.sum(v[0] * v[0], axis=-1)


def _argmax_body(e_ref, n_ref, s_ref):
    B, T = e_ref.shape
    R = n_ref.shape[0]
    e16 = jnp.broadcast_to(e_ref[...][:, None, :], (B, R // B, T)).reshape(R, T)
    v = e16 + n_ref[...]
    m = jnp.max(v, axis=-1, keepdims=True)
    ii = jax.lax.broadcasted_iota(jnp.int32, v.shape, 1)
    s_ref[0, :] = jnp.min(jnp.where(v == m, ii, T), axis=-1)


def _sc_shfl(v, idx):
    return lax.gather(
        v, idx[:, None],
        dimension_numbers=lax.GatherDimensionNumbers(
            offset_dims=(), collapsed_slice_dims=(0,), start_index_map=(0,)),
        slice_sizes=(1,), mode=lax.GatherScatterMode.PROMISE_IN_BOUNDS)


def _sc_argmax_body(e_hbm, n_hbm, s_hbm, e_v, n_v, o_v):
    T = 8192
    wid = lax.axis_index("c") * 16 + lax.axis_index("s")

    @pl.when(wid < 16)
    def _():
        b = wid // _NUM_MASKS
        pltpu.sync_copy(e_hbm.at[b], e_v)
        pltpu.sync_copy(n_hbm.at[wid], n_v)
        lanes = lax.iota(jnp.int32, 16)

        def step(j, carry):
            m, bj = carry
            v = e_v[pl.ds(j * 16, 16)] + n_v[pl.ds(j * 16, 16)]
            upd = v > m
            return jnp.where(upd, v, m), jnp.where(upd, j, bj)

        m0 = jnp.full((16,), -jnp.inf, jnp.float32)
        j0 = jnp.zeros((16,), jnp.int32)
        m, bj = lax.fori_loop(0, T // 16, step, (m0, j0), unroll=8)
        mx = m
        for k in (1, 2, 4, 8):
            mx = jnp.maximum(mx, _sc_shfl(mx, lanes ^ k))
        cand = jnp.where(m == mx, bj * 16 + lanes, T)
        for k in (1, 2, 4, 8):
            cand = jnp.minimum(cand, _sc_shfl(cand, lanes ^ k))
        o_v[...] = cand
        pltpu.sync_copy(o_v, s_hbm.at[wid])


def _mask_body(starts_ref, out_ref, o_ref):
    T = 8192
    b = pl.program_id(0)
    i = pl.program_id(1)
    j = pl.program_id(2)
    s_i = starts_ref[b * _NUM_MASKS + i, 0]
    row0 = jnp.minimum(s_i // _BLK + j, T // _BLK - 1) * _BLK
    rows = row0 + jax.lax.broadcasted_iota(jnp.int32, (_BLK, 1), 0)
    mask = jnp.zeros((_BLK, 1), jnp.bool_)
    for k in range(_NUM_MASKS):
        sk = starts_ref[b * _NUM_MASKS + k, 0]
        mask = mask | ((rows >= sk) & (rows < sk + _MASK_LEN))
    o_ref[...] = jnp.where(mask[None], jnp.float32(0.0), out_ref[...])


def _mask_index(b, i, j, starts):
    T = 8192
    row = jnp.minimum(starts[b * _NUM_MASKS + i, 0] // _BLK + j, T // _BLK - 1)
    return (b, row, 0)


@functools.partial(jax.jit, static_argnames=("interpret",))
def kernel(x, interpret=False):
    B, T, C = x.shape

    out, energy3 = pl.pallas_call(
        _copy_energy_body,
        grid=(B, T // _TT),
        in_specs=[pl.BlockSpec((1, _TT, C), lambda b, t: (b, t, 0))],
        out_specs=[pl.BlockSpec((1, _TT, C), lambda b, t: (b, t, 0)),
                   pl.BlockSpec((1, 1, _TT), lambda b, t: (b, 0, t))],
        out_shape=[jax.ShapeDtypeStruct((B, T, C), x.dtype),
                   jax.ShapeDtypeStruct((B, 1, T), jnp.float32)],
        interpret=interpret,
    )(x)
    energy = energy3.reshape(B, T)

    with jax.ensure_compile_time_eval():
        skey = jax.random.key(42)
        noise = jnp.stack(
            [jax.random.gumbel(jax.random.fold_in(skey, i), (B, T), jnp.float32)
             for i in range(_NUM_MASKS)], axis=1).reshape(B * _NUM_MASKS, T)

    if interpret:
        starts = pl.pallas_call(
            _argmax_body,
            out_shape=jax.ShapeDtypeStruct((1, B * _NUM_MASKS), jnp.int32),
            interpret=True,
        )(energy, noise)
        starts16 = jnp.broadcast_to(
            starts.reshape(B * _NUM_MASKS, 1), (B * _NUM_MASKS, 16))
    else:
        mesh = plsc.VectorSubcoreMesh(core_axis_name="c", subcore_axis_name="s")
        starts16 = pl.kernel(
            _sc_argmax_body,
            out_type=jax.ShapeDtypeStruct((B * _NUM_MASKS, 16), jnp.int32),
            mesh=mesh,
            scratch_types=[pltpu.VMEM((T,), jnp.float32),
                           pltpu.VMEM((T,), jnp.float32),
                           pltpu.VMEM((16,), jnp.int32)],
        )(energy, noise)

    out2 = pl.pallas_call(
        _mask_body,
        grid_spec=pltpu.PrefetchScalarGridSpec(
            num_scalar_prefetch=1,
            grid=(B, _NUM_MASKS, 2),
            in_specs=[pl.BlockSpec((1, _BLK, C), _mask_index)],
            out_specs=pl.BlockSpec((1, _BLK, C), _mask_index),
        ),
        out_shape=jax.ShapeDtypeStruct((B, T, C), x.dtype),
        input_output_aliases={1: 0},
        interpret=interpret,
    )(starts16, out)
    return out2


# speedup vs baseline: 1.0853x; 1.0096x over previous
import functools

import jax
import jax.numpy as jnp
from jax import lax
from jax.experimental import pallas as pl
from jax.experimental.pallas import tpu as pltpu
from jax.experimental.pallas import tpu_sc as plsc

_NUM_MASKS = 4
_MASK_LEN = 128
_BLK = 128
_TT = 4096


def _copy_energy_body(x_ref, out_ref, e_ref):
    v = x_ref[...]
    out_ref[...] = v
    e_ref[0, 0, :] = jnp.sum(v[0] * v[0], axis=-1)


def _argmax_body(e_ref, n_ref, s_ref):
    B, T = e_ref.shape
    R = n_ref.shape[0]
    e16 = jnp.broadcast_to(e_ref[...][:, None, :], (B, R // B, T)).reshape(R, T)
    v = e16 + n_ref[...]
    m = jnp.max(v, axis=-1, keepdims=True)
    ii = jax.lax.broadcasted_iota(jnp.int32, v.shape, 1)
    s_ref[0, :] = jnp.min(jnp.where(v == m, ii, T), axis=-1)


def _sc_shfl(v, idx):
    return lax.gather(
        v, idx[:, None],
        dimension_numbers=lax.GatherDimensionNumbers(
            offset_dims=(), collapsed_slice_dims=(0,), start_index_map=(0,)),
        slice_sizes=(1,), mode=lax.GatherScatterMode.PROMISE_IN_BOUNDS)


def _sc_argmax_body(e_hbm, n_hbm, s_hbm, e_v, n_v, o_v):
    T = 8192
    wid = lax.axis_index("c") * 16 + lax.axis_index("s")

    @pl.when(wid < 16)
    def _():
        b = wid // _NUM_MASKS
        pltpu.sync_copy(e_hbm.at[b], e_v)
        pltpu.sync_copy(n_hbm.at[wid], n_v)
        lanes = lax.iota(jnp.int32, 16)

        def step(j, carry):
            m, bj = carry
            v = e_v[pl.ds(j * 16, 16)] + n_v[pl.ds(j * 16, 16)]
            upd = v > m
            return jnp.where(upd, v, m), jnp.where(upd, j, bj)

        m0 = jnp.full((16,), -jnp.inf, jnp.float32)
        j0 = jnp.zeros((16,), jnp.int32)
        m, bj = lax.fori_loop(0, T // 16, step, (m0, j0), unroll=8)
        mx = m
        for k in (1, 2, 4, 8):
            mx = jnp.maximum(mx, _sc_shfl(mx, lanes ^ k))
        cand = jnp.where(m == mx, bj * 16 + lanes, T)
        for k in (1, 2, 4, 8):
            cand = jnp.minimum(cand, _sc_shfl(cand, lanes ^ k))
        o_v[...] = cand
        pltpu.sync_copy(o_v, s_hbm.at[wid])


def _mask_body(starts_ref, out_ref, o_ref):
    T = 8192
    b = pl.program_id(0)
    i = pl.program_id(1)
    j = pl.program_id(2)
    s_i = starts_ref[b * _NUM_MASKS + i, 0]
    row0 = jnp.minimum(s_i // _BLK + j, T // _BLK - 1) * _BLK
    rows = row0 + jax.lax.broadcasted_iota(jnp.int32, (_BLK, 1), 0)
    mask = jnp.zeros((_BLK, 1), jnp.bool_)
    for k in range(_NUM_MASKS):
        sk = starts_ref[b * _NUM_MASKS + k, 0]
        mask = mask | ((rows >= sk) & (rows < sk + _MASK_LEN))
    o_ref[...] = jnp.where(mask[None], jnp.float32(0.0), out_ref[...])


def _mask_index(b, i, j, starts):
    T = 8192
    row = jnp.minimum(starts[b * _NUM_MASKS + i, 0] // _BLK + j, T // _BLK - 1)
    return (b, row, 0)


@functools.partial(jax.jit, static_argnames=("interpret",))
def kernel(x, interpret=False):
    B, T, C = x.shape

    out, energy3 = pl.pallas_call(
        _copy_energy_body,
        grid=(B, T // _TT),
        in_specs=[pl.BlockSpec((1, _TT, C), lambda b, t: (b, t, 0))],
        out_specs=[pl.BlockSpec((1, _TT, C), lambda b, t: (b, t, 0)),
                   pl.BlockSpec((1, 1, _TT), lambda b, t: (b, 0, t))],
        out_shape=[jax.ShapeDtypeStruct((B, T, C), x.dtype),
                   jax.ShapeDtypeStruct((B, 1, T), jnp.float32)],
        interpret=interpret,
    )(x)
    energy = energy3.reshape(B, T)

    with jax.ensure_compile_time_eval():
        skey = jax.random.key(42)
        noise = jnp.stack(
            [jax.random.gumbel(jax.random.fold_in(skey, i), (B, T), jnp.float32)
             for i in range(_NUM_MASKS)], axis=1).reshape(B * _NUM_MASKS, T)

    if interpret:
        starts = pl.pallas_call(
            _argmax_body,
            out_shape=jax.ShapeDtypeStruct((1, B * _NUM_MASKS), jnp.int32),
            interpret=True,
        )(energy, noise)
        starts16 = jnp.broadcast_to(
            starts.reshape(B * _NUM_MASKS, 1), (B * _NUM_MASKS, 16))
    else:
        mesh = plsc.VectorSubcoreMesh(core_axis_name="c", subcore_axis_name="s")
        starts16 = pl.kernel(
            _sc_argmax_body,
            out_type=jax.ShapeDtypeStruct((B * _NUM_MASKS, 16), jnp.int32),
            mesh=mesh,
            scratch_types=[pltpu.VMEM((T,), jnp.float32),
                           pltpu.VMEM((T,), jnp.float32),
                           pltpu.VMEM((16,), jnp.int32)],
        )(energy, noise)

    out2 = pl.pallas_call(
        _mask_body,
        grid_spec=pltpu.PrefetchScalarGridSpec(
            num_scalar_prefetch=1,
            grid=(B, _NUM_MASKS, 2),
            in_specs=[pl.BlockSpec((1, _BLK, C), _mask_index)],
            out_specs=pl.BlockSpec((1, _BLK, C), _mask_index),
        ),
        out_shape=jax.ShapeDtypeStruct((B, T, C), x.dtype),
        input_output_aliases={1: 0},
        interpret=interpret,
    )(starts16, out)
    return out2
